# baseline (speedup 1.0000x reference)
import functools

import jax
import jax.numpy as jnp
from jax import lax
from jax.experimental import pallas as pl
from jax.experimental.pallas import tpu as pltpu
from jax.experimental.pallas import tpu_sc as plsc

R = 8192
S = 64
LANES = 16
CHUNKS = S // LANES
NC = 2
NS = 16
NW = NC * NS
SC_RAYS = 4096
TC_RAYS = R - SC_RAYS
TC_ROWS = TC_RAYS * S // 128
RPW = SC_RAYS // NW
LAMBDA_OPACITY = 0.001
LAMBDA_DISTORTION = 0.001

_mesh = plsc.VectorSubcoreMesh(core_axis_name="c", subcore_axis_name="s")


@functools.partial(
    pl.kernel,
    out_type=jax.ShapeDtypeStruct((SC_RAYS,), jnp.float32),
    mesh=_mesh,
    scratch_types=[
        pltpu.VMEM((RPW * S,), jnp.float32),
        pltpu.VMEM((RPW * S,), jnp.float32),
        pltpu.VMEM((RPW * S,), jnp.float32),
        pltpu.VMEM((RPW,), jnp.float32),
        pltpu.SemaphoreType.DMA,
        pltpu.SemaphoreType.DMA,
    ],
    compiler_params=pltpu.CompilerParams(
        needs_layout_passes=False, skip_device_barrier=True
    ),
)
def _distortion_sc(ws_hbm, deltas_hbm, ts_hbm, out_hbm, w_v, d_v, t_v, out_v,
                   sem0, sem1):
    wid = lax.axis_index("s") * NC + lax.axis_index("c")
    base = wid * (RPW * S)
    half = (RPW // 2) * S
    cps0 = [
        pltpu.make_async_copy(src.at[pl.ds(base, half)], dst.at[pl.ds(0, half)], sem0)
        for src, dst in ((ws_hbm, w_v), (deltas_hbm, d_v), (ts_hbm, t_v))
    ]
    cps1 = [
        pltpu.make_async_copy(
            src.at[pl.ds(base + half, half)], dst.at[pl.ds(half, half)], sem1
        )
        for src, dst in ((ws_hbm, w_v), (deltas_hbm, d_v), (ts_hbm, t_v))
    ]
    for cp in cps0:
        cp.start()
    for cp in cps1:
        cp.start()
    for cp in cps0:
        cp.wait()

    lane15 = lax.iota(jnp.int32, LANES) == (LANES - 1)

    def ray_work(r):
        idx15 = jnp.full((LANES,), LANES - 1, jnp.int32)
        ws_c = []
        ts_c = []
        cws = []
        for c in range(CHUNKS):
            off = r * S + c * LANES
            w = w_v[pl.ds(off, LANES)]
            t = t_v[pl.ds(off, LANES)]
            ws_c.append(w)
            ts_c.append(t)
            cws.append(plsc.cumsum(w))
        acc1 = jnp.zeros((LANES,), jnp.float32)
        acc2 = jnp.zeros((LANES,), jnp.float32)
        acc3 = jnp.zeros((LANES,), jnp.float32)
        accu = jnp.zeros((LANES,), jnp.float32)
        cw_carry = jnp.zeros((LANES,), jnp.float32)
        for c in range(CHUNKS):
            off = r * S + c * LANES
            w = ws_c[c]
            t = ts_c[c]
            d = d_v[pl.ds(off, LANES)]
            wt = w * t
            w_excl = (cws[c] - w) + cw_carry
            acc1 = acc1 + wt * w_excl
            acc2 = acc2 + wt * w
            acc3 = acc3 + wt
            accu = accu + (w * w) * d
            cw_carry = cw_carry + cws[c][idx15]
        final = (4.0 * acc1 + 2.0 * acc2 + (1.0 / 3.0) * accu
                 - (2.0 * cw_carry) * acc3)
        tot = plsc.cumsum(final) * LAMBDA_DISTORTION
        idx = jnp.full((LANES,), r, jnp.int32)
        plsc.store_scatter(out_v, [idx], tot, mask=lane15)

    plsc.parallel_loop(0, RPW // 2, 1)(ray_work)
    for cp in cps1:
        cp.wait()
    plsc.parallel_loop(RPW // 2, RPW, 1)(ray_work)
    pltpu.sync_copy(out_v, out_hbm.at[pl.ds(wid * RPW, RPW)])


def _rgb_opacity_tc(rgb_p_ref, rgb_t_ref, op_ref, drgb_ref, dop_ref):
    diff = rgb_p_ref[...] - rgb_t_ref[...]
    drgb_ref[...] = diff * diff
    o = op_ref[...] + 1e-10
    dop_ref[...] = (-LAMBDA_OPACITY) * o * jnp.log(o)


def _dist_tc(w_ref, t_ref, d_ref, out_ref):
    w = w_ref[...]
    t = t_ref[...]
    d = d_ref[...]
    i = lax.broadcasted_iota(jnp.int32, (128, 128), 0)
    j = lax.broadcasted_iota(jnp.int32, (128, 128), 1)
    samehalf = (i // S) == (j // S)
    ltri = jnp.where(samehalf & (i <= j), 1.0, 0.0).astype(jnp.float32)
    allh = jnp.where(samehalf, 1.0, 0.0).astype(jnp.float32)
    wincl = jnp.dot(w, ltri, preferred_element_type=jnp.float32)
    wtot = jnp.dot(w, allh, preferred_element_type=jnp.float32)
    wexcl = wincl - w
    wt = w * t
    fin = (4.0 * wt * wexcl + 2.0 * wt * w + (1.0 / 3.0) * (w * w) * d
           - (2.0 * wtot) * wt)
    hi = lax.broadcasted_iota(jnp.int32, (128, 2), 0) // S
    hj = lax.broadcasted_iota(jnp.int32, (128, 2), 1)
    bsel = jnp.where(hi == hj, 1.0, 0.0).astype(jnp.float32)
    out_ref[...] = jnp.dot(fin, bsel, preferred_element_type=jnp.float32) * (
        LAMBDA_DISTORTION
    )


def kernel(rgb_pred, rgb_target, opacity, ws, deltas, ts, rays_a):
    drgb_t, dop_t = pl.pallas_call(
        _rgb_opacity_tc,
        out_shape=(
            jax.ShapeDtypeStruct((3, R), jnp.float32),
            jax.ShapeDtypeStruct((1, R), jnp.float32),
        ),
    )(rgb_pred.T, rgb_target.T, opacity.T)
    d_sc = _distortion_sc(ws, deltas, ts)
    off = SC_RAYS * S
    d_tc = pl.pallas_call(
        _dist_tc,
        out_shape=jax.ShapeDtypeStruct((TC_ROWS, 2), jnp.float32),
    )(
        ws[off:].reshape(TC_ROWS, 128),
        ts[off:].reshape(TC_ROWS, 128),
        deltas[off:].reshape(TC_ROWS, 128),
    )
    d_distortion = jnp.concatenate([d_sc, d_tc.reshape(TC_RAYS)])
    return (drgb_t.T, dop_t.T, d_distortion)


# speedup vs baseline: 1.0352x; 1.0352x over previous
import functools

import jax
import jax.numpy as jnp
from jax import lax
from jax.experimental import pallas as pl
from jax.experimental.pallas import tpu as pltpu
from jax.experimental.pallas import tpu_sc as plsc

R = 8192
S = 64
LANES = 16
CHUNKS = S // LANES
NC = 2
NS = 16
NW = NC * NS
SC_RAYS = 4096
TC_RAYS = R - SC_RAYS
TC_ROWS = TC_RAYS * S // 128
RPW = SC_RAYS // NW
LAMBDA_OPACITY = 0.001
LAMBDA_DISTORTION = 0.001

_mesh = plsc.VectorSubcoreMesh(core_axis_name="c", subcore_axis_name="s")


@functools.partial(
    pl.kernel,
    out_type=jax.ShapeDtypeStruct((SC_RAYS,), jnp.float32),
    mesh=_mesh,
    scratch_types=[
        pltpu.VMEM((RPW * S,), jnp.float32),
        pltpu.VMEM((RPW * S,), jnp.float32),
        pltpu.VMEM((RPW * S,), jnp.float32),
        pltpu.VMEM((RPW,), jnp.float32),
        pltpu.SemaphoreType.DMA,
        pltpu.SemaphoreType.DMA,
    ],
    compiler_params=pltpu.CompilerParams(
        needs_layout_passes=False, skip_device_barrier=True
    ),
)
def _distortion_sc(ws_hbm, deltas_hbm, ts_hbm, out_hbm, w_v, d_v, t_v, out_v,
                   sem0, sem1):
    wid = lax.axis_index("s") * NC + lax.axis_index("c")
    base = wid * (RPW * S)
    half = (RPW // 2) * S
    cps0 = [
        pltpu.make_async_copy(src.at[pl.ds(base, half)], dst.at[pl.ds(0, half)], sem0)
        for src, dst in ((ws_hbm, w_v), (deltas_hbm, d_v), (ts_hbm, t_v))
    ]
    cps1 = [
        pltpu.make_async_copy(
            src.at[pl.ds(base + half, half)], dst.at[pl.ds(half, half)], sem1
        )
        for src, dst in ((ws_hbm, w_v), (deltas_hbm, d_v), (ts_hbm, t_v))
    ]
    for cp in cps0:
        cp.start()
    for cp in cps1:
        cp.start()
    for cp in cps0:
        cp.wait()

    lane15 = lax.iota(jnp.int32, LANES) == (LANES - 1)

    def ray_work(r):
        idx15 = jnp.full((LANES,), LANES - 1, jnp.int32)
        ws_c = []
        ts_c = []
        cws = []
        for c in range(CHUNKS):
            off = r * S + c * LANES
            w = w_v[pl.ds(off, LANES)]
            t = t_v[pl.ds(off, LANES)]
            ws_c.append(w)
            ts_c.append(t)
            cws.append(plsc.cumsum(w))
        acc1 = jnp.zeros((LANES,), jnp.float32)
        acc2 = jnp.zeros((LANES,), jnp.float32)
        acc3 = jnp.zeros((LANES,), jnp.float32)
        accu = jnp.zeros((LANES,), jnp.float32)
        cw_carry = jnp.zeros((LANES,), jnp.float32)
        for c in range(CHUNKS):
            off = r * S + c * LANES
            w = ws_c[c]
            t = ts_c[c]
            d = d_v[pl.ds(off, LANES)]
            wt = w * t
            w_excl = (cws[c] - w) + cw_carry
            acc1 = acc1 + wt * w_excl
            acc2 = acc2 + wt * w
            acc3 = acc3 + wt
            accu = accu + (w * w) * d
            cw_carry = cw_carry + cws[c][idx15]
        final = (4.0 * acc1 + 2.0 * acc2 + (1.0 / 3.0) * accu
                 - (2.0 * cw_carry) * acc3)
        tot = plsc.cumsum(final) * LAMBDA_DISTORTION
        idx = jnp.full((LANES,), r, jnp.int32)
        plsc.store_scatter(out_v, [idx], tot, mask=lane15)

    plsc.parallel_loop(0, RPW // 2, 1)(ray_work)
    for cp in cps1:
        cp.wait()
    plsc.parallel_loop(RPW // 2, RPW, 1)(ray_work)
    pltpu.sync_copy(out_v, out_hbm.at[pl.ds(wid * RPW, RPW)])


def _rgb_opacity_tc(rgb_p_ref, rgb_t_ref, op_ref, drgb_ref, dop_ref):
    diff = rgb_p_ref[...] - rgb_t_ref[...]
    drgb_ref[...] = diff * diff
    o = op_ref[...] + 1e-10
    dop_ref[...] = (-LAMBDA_OPACITY) * o * jnp.log(o)


def _dist_tc(w_ref, t_ref, d_ref, out_ref):
    w = w_ref[...]
    t = t_ref[...]
    d = d_ref[...]
    i = lax.broadcasted_iota(jnp.int32, (128, 128), 0)
    j = lax.broadcasted_iota(jnp.int32, (128, 128), 1)
    samehalf = (i // S) == (j // S)
    ltri = jnp.where(samehalf & (i <= j), 1.0, 0.0).astype(jnp.float32)
    allh = jnp.where(samehalf, 1.0, 0.0).astype(jnp.float32)
    wincl = jnp.dot(w, ltri, preferred_element_type=jnp.float32)
    wtot = jnp.dot(w, allh, preferred_element_type=jnp.float32)
    wexcl = wincl - w
    wt = w * t
    fin = (4.0 * wt * wexcl + 2.0 * wt * w + (1.0 / 3.0) * (w * w) * d
           - (2.0 * wtot) * wt)
    hi = lax.broadcasted_iota(jnp.int32, (128, 2), 0) // S
    hj = lax.broadcasted_iota(jnp.int32, (128, 2), 1)
    bsel = jnp.where(hi == hj, 1.0, 0.0).astype(jnp.float32)
    out_ref[...] = jnp.dot(fin, bsel, preferred_element_type=jnp.float32) * (
        LAMBDA_DISTORTION
    )


def kernel(rgb_pred, rgb_target, opacity, ws, deltas, ts, rays_a):
    drgb_t, dop_t = pl.pallas_call(
        _rgb_opacity_tc,
        out_shape=(
            jax.ShapeDtypeStruct((3, R), jnp.float32),
            jax.ShapeDtypeStruct((1, R), jnp.float32),
        ),
    )(rgb_pred.T, rgb_target.T, opacity.T)
    d_sc = _distortion_sc(ws, deltas, ts)
    blk = 256
    nblk = TC_ROWS // blk
    row0 = (SC_RAYS * S) // 128 // blk
    d_tc = pl.pallas_call(
        _dist_tc,
        grid=(nblk,),
        in_specs=[
            pl.BlockSpec((blk, 128), lambda i: (row0 + i, 0)),
            pl.BlockSpec((blk, 128), lambda i: (row0 + i, 0)),
            pl.BlockSpec((blk, 128), lambda i: (row0 + i, 0)),
        ],
        out_specs=pl.BlockSpec((blk, 2), lambda i: (i, 0)),
        out_shape=jax.ShapeDtypeStruct((TC_ROWS, 2), jnp.float32),
    )(
        ws.reshape(R * S // 128, 128),
        ts.reshape(R * S // 128, 128),
        deltas.reshape(R * S // 128, 128),
    )
    d_distortion = jnp.concatenate([d_sc, d_tc.reshape(TC_RAYS)])
    return (drgb_t.T, dop_t.T, d_distortion)


# speedup vs baseline: 1.1321x; 1.0937x over previous
import functools

import jax
import jax.numpy as jnp
from jax import lax
from jax.experimental import pallas as pl
from jax.experimental.pallas import tpu as pltpu
from jax.experimental.pallas import tpu_sc as plsc

R = 8192
S = 64
LANES = 16
CHUNKS = S // LANES
NC = 2
NS = 16
NW = NC * NS
RPW = R // NW
LAMBDA_OPACITY = 0.001
LAMBDA_DISTORTION = 0.001

_mesh = plsc.VectorSubcoreMesh(core_axis_name="c", subcore_axis_name="s")


@functools.partial(
    pl.kernel,
    out_type=jax.ShapeDtypeStruct((R,), jnp.float32),
    mesh=_mesh,
    scratch_types=[
        pltpu.VMEM((RPW * S,), jnp.float32),
        pltpu.VMEM((RPW * S,), jnp.float32),
        pltpu.VMEM((RPW * S,), jnp.float32),
        pltpu.VMEM((RPW,), jnp.float32),
        pltpu.SemaphoreType.DMA,
        pltpu.SemaphoreType.DMA,
    ],
    compiler_params=pltpu.CompilerParams(
        needs_layout_passes=False, skip_device_barrier=True
    ),
)
def _distortion_sc(ws_hbm, deltas_hbm, ts_hbm, out_hbm, w_v, d_v, t_v, out_v,
                   sem0, sem1):
    wid = lax.axis_index("s") * NC + lax.axis_index("c")
    base = wid * (RPW * S)
    half = (RPW // 2) * S
    cps0 = [
        pltpu.make_async_copy(src.at[pl.ds(base, half)], dst.at[pl.ds(0, half)], sem0)
        for src, dst in ((ws_hbm, w_v), (deltas_hbm, d_v), (ts_hbm, t_v))
    ]
    cps1 = [
        pltpu.make_async_copy(
            src.at[pl.ds(base + half, half)], dst.at[pl.ds(half, half)], sem1
        )
        for src, dst in ((ws_hbm, w_v), (deltas_hbm, d_v), (ts_hbm, t_v))
    ]
    for cp in cps0:
        cp.start()
    for cp in cps1:
        cp.start()
    for cp in cps0:
        cp.wait()

    lane15 = lax.iota(jnp.int32, LANES) == (LANES - 1)

    def ray_work(r):
        idx15 = jnp.full((LANES,), LANES - 1, jnp.int32)
        ws_c = []
        ts_c = []
        cws = []
        for c in range(CHUNKS):
            off = r * S + c * LANES
            w = w_v[pl.ds(off, LANES)]
            t = t_v[pl.ds(off, LANES)]
            ws_c.append(w)
            ts_c.append(t)
            cws.append(plsc.cumsum(w))
        acc1 = jnp.zeros((LANES,), jnp.float32)
        acc2 = jnp.zeros((LANES,), jnp.float32)
        acc3 = jnp.zeros((LANES,), jnp.float32)
        accu = jnp.zeros((LANES,), jnp.float32)
        cw_carry = jnp.zeros((LANES,), jnp.float32)
        for c in range(CHUNKS):
            off = r * S + c * LANES
            w = ws_c[c]
            t = ts_c[c]
            d = d_v[pl.ds(off, LANES)]
            wt = w * t
            w_excl = (cws[c] - w) + cw_carry
            acc1 = acc1 + wt * w_excl
            acc2 = acc2 + wt * w
            acc3 = acc3 + wt
            accu = accu + (w * w) * d
            cw_carry = cw_carry + cws[c][idx15]
        final = (4.0 * acc1 + 2.0 * acc2 + (1.0 / 3.0) * accu
                 - (2.0 * cw_carry) * acc3)
        tot = plsc.cumsum(final) * LAMBDA_DISTORTION
        idx = jnp.full((LANES,), r, jnp.int32)
        plsc.store_scatter(out_v, [idx], tot, mask=lane15)

    plsc.parallel_loop(0, RPW // 2, 1)(ray_work)
    for cp in cps1:
        cp.wait()
    plsc.parallel_loop(RPW // 2, RPW, 1)(ray_work)
    pltpu.sync_copy(out_v, out_hbm.at[pl.ds(wid * RPW, RPW)])


def _rgb_opacity_tc(rgb_p_ref, rgb_t_ref, op_ref, drgb_ref, dop_ref):
    diff = rgb_p_ref[...] - rgb_t_ref[...]
    drgb_ref[...] = diff * diff
    o = op_ref[...] + 1e-10
    dop_ref[...] = (-LAMBDA_OPACITY) * o * jnp.log(o)


def kernel(rgb_pred, rgb_target, opacity, ws, deltas, ts, rays_a):
    drgb_t, dop_t = pl.pallas_call(
        _rgb_opacity_tc,
        out_shape=(
            jax.ShapeDtypeStruct((3, R), jnp.float32),
            jax.ShapeDtypeStruct((1, R), jnp.float32),
        ),
    )(rgb_pred.T, rgb_target.T, opacity.T)
    d_distortion = _distortion_sc(ws, deltas, ts)
    return (drgb_t.T, dop_t.T, d_distortion)
